# initial kernel scaffold (unmeasured)
import jax
import jax.numpy as jnp
from jax import lax
from jax.experimental import pallas as pl
from jax.experimental.pallas import tpu as pltpu

N_DEV = 4
SQ = 2048
SKV = 2048
HQ_LOCAL = 8
DH = 128
D_MODEL = 1024
D_LOCAL = HQ_LOCAL * DH
SCALE = 0.08838834764831843
BLK = 64


def kernel(x, Wq, K_ext, V_ext, Wo):
    my = lax.axis_index("i")
    xs = x[0].astype(jnp.bfloat16)
    Wq_l = lax.dynamic_slice_in_dim(Wq, my * D_LOCAL, D_LOCAL, axis=1)
    Wq_l = Wq_l.astype(jnp.bfloat16)
    Wo_l = lax.dynamic_slice_in_dim(Wo, my * D_LOCAL, D_LOCAL, axis=0)
    Wo_l = Wo_l.astype(jnp.bfloat16)
    K2 = K_ext[0].reshape(SKV, D_LOCAL).astype(jnp.bfloat16)
    V2 = V_ext[0].reshape(SKV, D_LOCAL).astype(jnp.bfloat16)

    def body(x_ref, wq_ref, k_ref, v_ref, wo_ref, out_ref,
             ctx_ref, comm_ref, send_sems, recv_sems):
        my_pos = lax.axis_index("i")
        left = lax.rem(my_pos + N_DEV - 1, N_DEV)
        right = lax.rem(my_pos + 1, N_DEV)

        barrier_sem = pltpu.get_barrier_semaphore()
        for nbr in (left, right):
            pl.semaphore_signal(
                barrier_sem, inc=1,
                device_id=(nbr,), device_id_type=pl.DeviceIdType.MESH,
            )
        pl.semaphore_wait(barrier_sem, 2)

        qb = lax.broadcasted_iota(jnp.int32, (SQ, 1), 0) // BLK
        kb = lax.broadcasted_iota(jnp.int32, (1, SKV), 1) // BLK
        keep = kb <= qb

        def head_body(h, carry):
            c = pl.ds(h * DH, DH)
            q = jnp.dot(x_ref[...], wq_ref[:, c],
                        preferred_element_type=jnp.float32)
            q = (q * SCALE).astype(jnp.bfloat16)
            k = k_ref[:, c]
            s = lax.dot_general(q, k, (((1,), (1,)), ((), ())),
                                preferred_element_type=jnp.float32)
            s = jnp.where(keep, s, -1e9)
            m = jnp.max(s, axis=1, keepdims=True)
            w = jnp.exp(s - m)
            w = (w / jnp.sum(w, axis=1, keepdims=True)).astype(jnp.bfloat16)
            ctx_ref[:, c] = jnp.dot(
                w, v_ref[:, c], preferred_element_type=jnp.float32
            ).astype(jnp.bfloat16)
            return carry

        lax.fori_loop(0, HQ_LOCAL, head_body, 0)

        partial = jnp.dot(ctx_ref[...], wo_ref[...],
                          preferred_element_type=jnp.float32)
        out_ref[...] = partial
        comm_ref[0, :, :] = partial.astype(jnp.bfloat16)

        for hop in range(N_DEV - 1):
            rdma = pltpu.make_async_remote_copy(
                src_ref=comm_ref.at[hop],
                dst_ref=comm_ref.at[hop + 1],
                send_sem=send_sems.at[hop],
                recv_sem=recv_sems.at[hop],
                device_id=(right,),
                device_id_type=pl.DeviceIdType.MESH,
            )
            rdma.start()
            rdma.wait()
            out_ref[...] += comm_ref[hop + 1].astype(jnp.float32)

    out = pl.pallas_call(
        body,
        out_shape=jax.ShapeDtypeStruct((SQ, D_MODEL), jnp.float32),
        in_specs=[pl.BlockSpec(memory_space=pltpu.VMEM)] * 5,
        out_specs=pl.BlockSpec(memory_space=pltpu.VMEM),
        scratch_shapes=[
            pltpu.VMEM((SQ, D_LOCAL), jnp.bfloat16),
            pltpu.VMEM((N_DEV, SQ, D_MODEL), jnp.bfloat16),
            pltpu.SemaphoreType.DMA((N_DEV - 1,)),
            pltpu.SemaphoreType.DMA((N_DEV - 1,)),
        ],
        compiler_params=pltpu.CompilerParams(collective_id=0),
    )(xs, Wq_l, K2, V2, Wo_l)
    return out[None]


# baseline (device time: 276362 ns/iter reference)
import jax
import jax.numpy as jnp
from jax import lax
from jax.experimental import pallas as pl
from jax.experimental.pallas import tpu as pltpu

N_DEV = 4
SQ = 2048
SKV = 2048
HQ_LOCAL = 8
DH = 128
D_MODEL = 1024
D_LOCAL = HQ_LOCAL * DH
SCALE = 0.08838834764831843
BLK = 64


def kernel(x, Wq, K_ext, V_ext, Wo):
    my = lax.axis_index("i")
    xs = x[0].astype(jnp.bfloat16)
    Wq_l = lax.dynamic_slice_in_dim(Wq, my * D_LOCAL, D_LOCAL, axis=1)
    Wq_l = Wq_l.astype(jnp.bfloat16)
    Wo_l = lax.dynamic_slice_in_dim(Wo, my * D_LOCAL, D_LOCAL, axis=0)
    Wo_l = Wo_l.astype(jnp.bfloat16)
    K2 = K_ext[0].reshape(SKV, D_LOCAL).astype(jnp.bfloat16)
    V2 = V_ext[0].reshape(SKV, D_LOCAL).astype(jnp.bfloat16)

    def body(x_ref, wq_ref, k_ref, v_ref, wo_ref, out_ref,
             ctx_ref, comm_ref, send_sems, recv_sems):
        my_pos = lax.axis_index("i")
        left = lax.rem(my_pos + N_DEV - 1, N_DEV)
        right = lax.rem(my_pos + 1, N_DEV)

        barrier_sem = pltpu.get_barrier_semaphore()
        for nbr in (left, right):
            pl.semaphore_signal(
                barrier_sem, inc=1,
                device_id=(nbr,), device_id_type=pl.DeviceIdType.MESH,
            )
        pl.semaphore_wait(barrier_sem, 2)

        qb = lax.broadcasted_iota(jnp.int32, (SQ, 1), 0) // BLK
        kb = lax.broadcasted_iota(jnp.int32, (1, SKV), 1) // BLK
        keep = kb <= qb

        def head_body(h, carry):
            c = pl.ds(h * DH, DH)
            q = jnp.dot(x_ref[...], wq_ref[:, c],
                        preferred_element_type=jnp.float32)
            q = (q * SCALE).astype(jnp.bfloat16)
            k = k_ref[:, c]
            s = lax.dot_general(q, k, (((1,), (1,)), ((), ())),
                                preferred_element_type=jnp.float32)
            s = jnp.where(keep, s, -1e9)
            m = jnp.max(s, axis=1, keepdims=True)
            w = jnp.exp(s - m)
            w = (w / jnp.sum(w, axis=1, keepdims=True)).astype(jnp.bfloat16)
            ctx_ref[:, c] = jnp.dot(
                w, v_ref[:, c], preferred_element_type=jnp.float32
            ).astype(jnp.bfloat16)
            return carry

        lax.fori_loop(0, HQ_LOCAL, head_body, 0)

        partial = jnp.dot(ctx_ref[...], wo_ref[...],
                          preferred_element_type=jnp.float32)
        out_ref[...] = partial
        comm_ref[0, :, :] = partial.astype(jnp.bfloat16)

        for hop in range(N_DEV - 1):
            rdma = pltpu.make_async_remote_copy(
                src_ref=comm_ref.at[hop],
                dst_ref=comm_ref.at[hop + 1],
                send_sem=send_sems.at[hop],
                recv_sem=recv_sems.at[hop],
                device_id=(right,),
                device_id_type=pl.DeviceIdType.MESH,
            )
            rdma.start()
            rdma.wait()
            out_ref[...] += comm_ref[hop + 1].astype(jnp.float32)

    out = pl.pallas_call(
        body,
        out_shape=jax.ShapeDtypeStruct((SQ, D_MODEL), jnp.float32),
        in_specs=[pl.BlockSpec(memory_space=pltpu.VMEM)] * 5,
        out_specs=pl.BlockSpec(memory_space=pltpu.VMEM),
        scratch_shapes=[
            pltpu.VMEM((SQ, D_LOCAL), jnp.bfloat16),
            pltpu.VMEM((N_DEV, SQ, D_MODEL), jnp.bfloat16),
            pltpu.SemaphoreType.DMA((N_DEV - 1,)),
            pltpu.SemaphoreType.DMA((N_DEV - 1,)),
        ],
        compiler_params=pltpu.CompilerParams(
            collective_id=0,
            vmem_limit_bytes=100 * 1024 * 1024,
        ),
    )(xs, Wq_l, K2, V2, Wo_l)
    return out[None]


# device time: 190334 ns/iter; 1.4520x vs baseline; 1.4520x over previous
import jax
import jax.numpy as jnp
from jax import lax
from jax.experimental import pallas as pl
from jax.experimental.pallas import tpu as pltpu

N_DEV = 4
SQ = 2048
SKV = 2048
HQ_LOCAL = 8
DH = 128
D_MODEL = 1024
D_LOCAL = HQ_LOCAL * DH
SCALE = 0.08838834764831843
BLK = 64
CH = SQ // N_DEV


def kernel(x, Wq, K_ext, V_ext, Wo):
    my = lax.axis_index("i")
    xs = x[0].astype(jnp.bfloat16)
    Wq_l = lax.dynamic_slice_in_dim(Wq, my * D_LOCAL, D_LOCAL, axis=1)
    Wq_l = Wq_l.astype(jnp.bfloat16)
    Wo_l = lax.dynamic_slice_in_dim(Wo, my * D_LOCAL, D_LOCAL, axis=0)
    Wo_l = Wo_l.astype(jnp.bfloat16)
    K2 = K_ext[0].reshape(SKV, D_LOCAL).astype(jnp.bfloat16)
    V2 = V_ext[0].reshape(SKV, D_LOCAL).astype(jnp.bfloat16)

    def body(x_ref, wq_ref, k_ref, v_ref, wo_ref, out_ref,
             ctx_ref, q_ref, rs_send_ref, rs_recv_ref, ag_ref,
             rs_send_sems, rs_recv_sems, ag_send_sems, ag_recv_sems):
        my_pos = lax.axis_index("i")
        left = lax.rem(my_pos + N_DEV - 1, N_DEV)
        right = lax.rem(my_pos + 1, N_DEV)

        barrier_sem = pltpu.get_barrier_semaphore()
        for nbr in (left, right):
            pl.semaphore_signal(
                barrier_sem, inc=1,
                device_id=(nbr,), device_id_type=pl.DeviceIdType.MESH,
            )
        pl.semaphore_wait(barrier_sem, 2)

        for r in range(N_DEV):
            rows = pl.ds(r * CH, CH)
            nk = (r + 1) * CH
            q_all = jnp.dot(x_ref[rows, :], wq_ref[...],
                            preferred_element_type=jnp.float32)
            q_ref[...] = (q_all * SCALE).astype(jnp.bfloat16)
            qb = lax.broadcasted_iota(jnp.int32, (CH, 1), 0) // BLK \
                + r * (CH // BLK)
            kb = lax.broadcasted_iota(jnp.int32, (1, nk), 1) // BLK
            keep = kb <= qb

            def head_body(h, carry, r=r, nk=nk, keep=keep):
                c = pl.ds(h * DH, DH)
                s = lax.dot_general(q_ref[:, c], k_ref[pl.ds(0, nk), c],
                                    (((1,), (1,)), ((), ())),
                                    preferred_element_type=jnp.float32)
                s = jnp.where(keep, s, -1e9)
                m = jnp.max(s, axis=1, keepdims=True)
                w = jnp.exp(s - m)
                w = (w / jnp.sum(w, axis=1, keepdims=True)).astype(jnp.bfloat16)
                ctx_ref[pl.ds(r * CH, CH), c] = jnp.dot(
                    w, v_ref[pl.ds(0, nk), c],
                    preferred_element_type=jnp.float32,
                ).astype(jnp.bfloat16)
                return carry

            lax.fori_loop(0, HQ_LOCAL, head_body, 0)
            out_ref[rows, :] = jnp.dot(ctx_ref[rows, :], wo_ref[...],
                                       preferred_element_type=jnp.float32)

        for t in range(N_DEV - 1):
            s_idx = lax.rem(my_pos + 2 * N_DEV - 1 - t, N_DEV)
            rs_send_ref[t, :, :] = out_ref[pl.ds(s_idx * CH, CH), :].astype(
                jnp.bfloat16)
            rdma = pltpu.make_async_remote_copy(
                src_ref=rs_send_ref.at[t],
                dst_ref=rs_recv_ref.at[t],
                send_sem=rs_send_sems.at[t],
                recv_sem=rs_recv_sems.at[t],
                device_id=(right,),
                device_id_type=pl.DeviceIdType.MESH,
            )
            rdma.start()
            rdma.wait()
            r_idx = lax.rem(my_pos + 2 * N_DEV - 2 - t, N_DEV)
            out_ref[pl.ds(r_idx * CH, CH), :] += rs_recv_ref[t].astype(
                jnp.float32)

        ag_ref[0, :, :] = out_ref[pl.ds(my_pos * CH, CH), :].astype(
            jnp.bfloat16)
        for h in range(N_DEV - 1):
            rdma = pltpu.make_async_remote_copy(
                src_ref=ag_ref.at[h],
                dst_ref=ag_ref.at[h + 1],
                send_sem=ag_send_sems.at[h],
                recv_sem=ag_recv_sems.at[h],
                device_id=(right,),
                device_id_type=pl.DeviceIdType.MESH,
            )
            rdma.start()
            rdma.wait()
            o_idx = lax.rem(my_pos + 2 * N_DEV - 1 - h, N_DEV)
            out_ref[pl.ds(o_idx * CH, CH), :] = ag_ref[h + 1].astype(
                jnp.float32)

    out = pl.pallas_call(
        body,
        out_shape=jax.ShapeDtypeStruct((SQ, D_MODEL), jnp.float32),
        in_specs=[pl.BlockSpec(memory_space=pltpu.VMEM)] * 5,
        out_specs=pl.BlockSpec(memory_space=pltpu.VMEM),
        scratch_shapes=[
            pltpu.VMEM((SQ, D_LOCAL), jnp.bfloat16),
            pltpu.VMEM((CH, D_LOCAL), jnp.bfloat16),
            pltpu.VMEM((N_DEV - 1, CH, D_MODEL), jnp.bfloat16),
            pltpu.VMEM((N_DEV - 1, CH, D_MODEL), jnp.bfloat16),
            pltpu.VMEM((N_DEV, CH, D_MODEL), jnp.bfloat16),
            pltpu.SemaphoreType.DMA((N_DEV - 1,)),
            pltpu.SemaphoreType.DMA((N_DEV - 1,)),
            pltpu.SemaphoreType.DMA((N_DEV - 1,)),
            pltpu.SemaphoreType.DMA((N_DEV - 1,)),
        ],
        compiler_params=pltpu.CompilerParams(
            collective_id=0,
            vmem_limit_bytes=100 * 1024 * 1024,
        ),
    )(xs, Wq_l, K2, V2, Wo_l)
    return out[None]


# device time: 152293 ns/iter; 1.8147x vs baseline; 1.2498x over previous
import jax
import jax.numpy as jnp
from jax import lax
from jax.experimental import pallas as pl
from jax.experimental.pallas import tpu as pltpu

N_DEV = 4
SQ = 2048
SKV = 2048
HQ_LOCAL = 8
DH = 128
D_MODEL = 1024
D_LOCAL = HQ_LOCAL * DH
SCALE = 0.08838834764831843
BLK = 64
CH = SQ // N_DEV
QC = D_MODEL // N_DEV


def kernel(x, Wq, K_ext, V_ext, Wo):
    my = lax.axis_index("i")
    xs = x[0].astype(jnp.bfloat16)
    Wq_l = lax.dynamic_slice_in_dim(Wq, my * D_LOCAL, D_LOCAL, axis=1)
    Wq_l = Wq_l.astype(jnp.bfloat16)
    Wo_l = lax.dynamic_slice_in_dim(Wo, my * D_LOCAL, D_LOCAL, axis=0)
    Wo_l = Wo_l.astype(jnp.bfloat16)
    K2 = K_ext[0].reshape(SKV, D_LOCAL).astype(jnp.bfloat16)
    V2 = V_ext[0].reshape(SKV, D_LOCAL).astype(jnp.bfloat16)

    def body(x_ref, wq_ref, k_ref, v_ref, wo_ref, out_ref,
             ctx_ref, q_ref, rs_send_ref, rs_recv_ref, ag_send_ref,
             ag_recv_ref, rs_send_sems, rs_recv_sems, ag_send_sems,
             ag_recv_sems):
        my_pos = lax.axis_index("i")
        peers = [lax.rem(my_pos + 1 + p, N_DEV) for p in range(N_DEV - 1)]

        barrier_sem = pltpu.get_barrier_semaphore()
        for pr in peers:
            pl.semaphore_signal(
                barrier_sem, inc=1,
                device_id=(pr,), device_id_type=pl.DeviceIdType.MESH,
            )
        pl.semaphore_wait(barrier_sem, N_DEV - 1)

        my_cols = pl.ds(my_pos * QC, QC)
        pending_sends = []

        for r in range(N_DEV):
            rows = pl.ds(r * CH, CH)
            nk = (r + 1) * CH
            q_all = jnp.dot(x_ref[rows, :], wq_ref[...],
                            preferred_element_type=jnp.float32)
            q_ref[...] = (q_all * SCALE).astype(jnp.bfloat16)
            qb = lax.broadcasted_iota(jnp.int32, (CH, 1), 0) // BLK \
                + r * (CH // BLK)
            kb = lax.broadcasted_iota(jnp.int32, (1, nk), 1) // BLK
            keep = kb <= qb

            def head_body(h, carry, r=r, nk=nk, keep=keep):
                c = pl.ds(h * DH, DH)
                s = lax.dot_general(q_ref[:, c], k_ref[pl.ds(0, nk), c],
                                    (((1,), (1,)), ((), ())),
                                    preferred_element_type=jnp.float32)
                s = jnp.where(keep, s, -1e9)
                m = jnp.max(s, axis=1, keepdims=True)
                w = jnp.exp(s - m)
                w = (w / jnp.sum(w, axis=1, keepdims=True)).astype(jnp.bfloat16)
                ctx_ref[pl.ds(r * CH, CH), c] = jnp.dot(
                    w, v_ref[pl.ds(0, nk), c],
                    preferred_element_type=jnp.float32,
                ).astype(jnp.bfloat16)
                return carry

            lax.fori_loop(0, HQ_LOCAL, head_body, 0)
            partial_r = jnp.dot(ctx_ref[rows, :], wo_ref[...],
                                preferred_element_type=jnp.float32)
            out_ref[rows, :] = partial_r
            rs_send_ref[r, :, :] = partial_r.astype(jnp.bfloat16)

            for p in range(N_DEV - 1):
                tgt = peers[p]
                rdma = pltpu.make_async_remote_copy(
                    src_ref=rs_send_ref.at[r, :, pl.ds(tgt * QC, QC)],
                    dst_ref=rs_recv_ref.at[r, 2 - p],
                    send_sem=rs_send_sems.at[r * 3 + p],
                    recv_sem=rs_recv_sems.at[r * 3 + (2 - p)],
                    device_id=(tgt,),
                    device_id_type=pl.DeviceIdType.MESH,
                )
                rdma.start()
                pending_sends.append(rdma)

            acc = out_ref[rows, my_cols]
            for q in range(N_DEV - 1):
                recv = pltpu.make_async_remote_copy(
                    src_ref=rs_recv_ref.at[r, q],
                    dst_ref=rs_recv_ref.at[r, q],
                    send_sem=rs_send_sems.at[r * 3 + q],
                    recv_sem=rs_recv_sems.at[r * 3 + q],
                    device_id=(my_pos,),
                    device_id_type=pl.DeviceIdType.MESH,
                )
                recv.wait_recv()
                acc = acc + rs_recv_ref[r, q].astype(jnp.float32)
            out_ref[rows, my_cols] = acc
            ag_send_ref[r, :, :] = acc.astype(jnp.bfloat16)
            for p in range(N_DEV - 1):
                tgt = peers[p]
                rdma = pltpu.make_async_remote_copy(
                    src_ref=ag_send_ref.at[r],
                    dst_ref=ag_recv_ref.at[r, 2 - p],
                    send_sem=ag_send_sems.at[r * 3 + p],
                    recv_sem=ag_recv_sems.at[r * 3 + (2 - p)],
                    device_id=(tgt,),
                    device_id_type=pl.DeviceIdType.MESH,
                )
                rdma.start()
                pending_sends.append(rdma)

        for r in range(N_DEV):
            rows = pl.ds(r * CH, CH)
            for q in range(N_DEV - 1):
                recv = pltpu.make_async_remote_copy(
                    src_ref=ag_recv_ref.at[r, q],
                    dst_ref=ag_recv_ref.at[r, q],
                    send_sem=ag_send_sems.at[r * 3 + q],
                    recv_sem=ag_recv_sems.at[r * 3 + q],
                    device_id=(my_pos,),
                    device_id_type=pl.DeviceIdType.MESH,
                )
                recv.wait_recv()
                out_ref[rows, pl.ds(peers[q] * QC, QC)] = (
                    ag_recv_ref[r, q].astype(jnp.float32))

        for rdma in pending_sends:
            rdma.wait_send()

    out = pl.pallas_call(
        body,
        out_shape=jax.ShapeDtypeStruct((SQ, D_MODEL), jnp.float32),
        in_specs=[pl.BlockSpec(memory_space=pltpu.VMEM)] * 5,
        out_specs=pl.BlockSpec(memory_space=pltpu.VMEM),
        scratch_shapes=[
            pltpu.VMEM((SQ, D_LOCAL), jnp.bfloat16),
            pltpu.VMEM((CH, D_LOCAL), jnp.bfloat16),
            pltpu.VMEM((N_DEV, CH, D_MODEL), jnp.bfloat16),
            pltpu.VMEM((N_DEV, N_DEV - 1, CH, QC), jnp.bfloat16),
            pltpu.VMEM((N_DEV, CH, QC), jnp.bfloat16),
            pltpu.VMEM((N_DEV, N_DEV - 1, CH, QC), jnp.bfloat16),
            pltpu.SemaphoreType.DMA((N_DEV * 3,)),
            pltpu.SemaphoreType.DMA((N_DEV * 3,)),
            pltpu.SemaphoreType.DMA((N_DEV * 3,)),
            pltpu.SemaphoreType.DMA((N_DEV * 3,)),
        ],
        compiler_params=pltpu.CompilerParams(
            collective_id=0,
            vmem_limit_bytes=100 * 1024 * 1024,
        ),
    )(xs, Wq_l, K2, V2, Wo_l)
    return out[None]


# device time: 118724 ns/iter; 2.3278x vs baseline; 1.2827x over previous
import jax
import jax.numpy as jnp
from jax import lax
from jax.experimental import pallas as pl
from jax.experimental.pallas import tpu as pltpu

N_DEV = 4
SQ = 2048
SKV = 2048
HQ_LOCAL = 8
DH = 128
D_MODEL = 1024
D_LOCAL = HQ_LOCAL * DH
SCALE = 0.08838834764831843
BLK = 64
CH = SQ // N_DEV
QC = D_MODEL // N_DEV


def kernel(x, Wq, K_ext, V_ext, Wo):
    my = lax.axis_index("i")
    xs = x[0].astype(jnp.bfloat16)
    Wq_l = lax.dynamic_slice_in_dim(Wq, my * D_LOCAL, D_LOCAL, axis=1)
    Wq_l = Wq_l.astype(jnp.bfloat16)
    Wo_l = lax.dynamic_slice_in_dim(Wo, my * D_LOCAL, D_LOCAL, axis=0)
    Wo_l = Wo_l.astype(jnp.bfloat16)
    K2 = K_ext[0].reshape(SKV, D_LOCAL).astype(jnp.bfloat16)
    V2 = V_ext[0].reshape(SKV, D_LOCAL).astype(jnp.bfloat16)

    def body(x_ref, wq_ref, k_ref, v_ref, wo_ref, out_ref,
             ctx_ref, q_ref, rs_send_ref, rs_recv_ref, ag_send_ref,
             ag_recv_ref, rs_send_sems, rs_recv_sems, ag_send_sems,
             ag_recv_sems):
        my_pos = lax.axis_index("i")
        peers = [lax.rem(my_pos + 1 + p, N_DEV) for p in range(N_DEV - 1)]

        barrier_sem = pltpu.get_barrier_semaphore()
        for pr in peers:
            pl.semaphore_signal(
                barrier_sem, inc=1,
                device_id=(pr,), device_id_type=pl.DeviceIdType.MESH,
            )
        pl.semaphore_wait(barrier_sem, N_DEV - 1)

        my_cols = pl.ds(my_pos * QC, QC)
        pending_sends = []

        for r in range(N_DEV):
            rows = pl.ds(r * CH, CH)
            nk = (r + 1) * CH
            q_all = jnp.dot(x_ref[rows, :], wq_ref[...],
                            preferred_element_type=jnp.float32)
            q_ref[...] = (q_all * SCALE).astype(jnp.bfloat16)
            qb = lax.broadcasted_iota(jnp.int32, (CH, 1), 0) // BLK \
                + r * (CH // BLK)
            kb = lax.broadcasted_iota(jnp.int32, (1, nk), 1) // BLK
            keep = kb <= qb

            def head_body(h, carry, r=r, nk=nk, keep=keep):
                c = pl.ds(h * DH, DH)
                s = lax.dot_general(q_ref[:, c], k_ref[pl.ds(0, nk), c],
                                    (((1,), (1,)), ((), ())),
                                    preferred_element_type=jnp.float32)
                w = jnp.exp(jnp.where(keep, s, -30.0))
                denom = jnp.sum(w, axis=1, keepdims=True)
                ctx = jnp.dot(w.astype(jnp.bfloat16), v_ref[pl.ds(0, nk), c],
                              preferred_element_type=jnp.float32)
                ctx_ref[pl.ds(r * CH, CH), c] = (ctx / denom).astype(
                    jnp.bfloat16)
                return carry

            lax.fori_loop(0, HQ_LOCAL, head_body, 0, unroll=2)
            partial_r = jnp.dot(ctx_ref[rows, :], wo_ref[...],
                                preferred_element_type=jnp.float32)
            out_ref[rows, :] = partial_r
            rs_send_ref[r, :, :] = partial_r.astype(jnp.bfloat16)

            for p in range(N_DEV - 1):
                tgt = peers[p]
                rdma = pltpu.make_async_remote_copy(
                    src_ref=rs_send_ref.at[r, :, pl.ds(tgt * QC, QC)],
                    dst_ref=rs_recv_ref.at[r, 2 - p],
                    send_sem=rs_send_sems.at[r * 3 + p],
                    recv_sem=rs_recv_sems.at[r * 3 + (2 - p)],
                    device_id=(tgt,),
                    device_id_type=pl.DeviceIdType.MESH,
                )
                rdma.start()
                pending_sends.append(rdma)

            acc = out_ref[rows, my_cols]
            for q in range(N_DEV - 1):
                recv = pltpu.make_async_remote_copy(
                    src_ref=rs_recv_ref.at[r, q],
                    dst_ref=rs_recv_ref.at[r, q],
                    send_sem=rs_send_sems.at[r * 3 + q],
                    recv_sem=rs_recv_sems.at[r * 3 + q],
                    device_id=(my_pos,),
                    device_id_type=pl.DeviceIdType.MESH,
                )
                recv.wait_recv()
                acc = acc + rs_recv_ref[r, q].astype(jnp.float32)
            out_ref[rows, my_cols] = acc
            ag_send_ref[r, :, :] = acc.astype(jnp.bfloat16)
            for p in range(N_DEV - 1):
                tgt = peers[p]
                rdma = pltpu.make_async_remote_copy(
                    src_ref=ag_send_ref.at[r],
                    dst_ref=ag_recv_ref.at[r, 2 - p],
                    send_sem=ag_send_sems.at[r * 3 + p],
                    recv_sem=ag_recv_sems.at[r * 3 + (2 - p)],
                    device_id=(tgt,),
                    device_id_type=pl.DeviceIdType.MESH,
                )
                rdma.start()
                pending_sends.append(rdma)

        for r in range(N_DEV):
            rows = pl.ds(r * CH, CH)
            for q in range(N_DEV - 1):
                recv = pltpu.make_async_remote_copy(
                    src_ref=ag_recv_ref.at[r, q],
                    dst_ref=ag_recv_ref.at[r, q],
                    send_sem=ag_send_sems.at[r * 3 + q],
                    recv_sem=ag_recv_sems.at[r * 3 + q],
                    device_id=(my_pos,),
                    device_id_type=pl.DeviceIdType.MESH,
                )
                recv.wait_recv()
                out_ref[rows, pl.ds(peers[q] * QC, QC)] = (
                    ag_recv_ref[r, q].astype(jnp.float32))

        for rdma in pending_sends:
            rdma.wait_send()

    out = pl.pallas_call(
        body,
        out_shape=jax.ShapeDtypeStruct((SQ, D_MODEL), jnp.float32),
        in_specs=[pl.BlockSpec(memory_space=pltpu.VMEM)] * 5,
        out_specs=pl.BlockSpec(memory_space=pltpu.VMEM),
        scratch_shapes=[
            pltpu.VMEM((SQ, D_LOCAL), jnp.bfloat16),
            pltpu.VMEM((CH, D_LOCAL), jnp.bfloat16),
            pltpu.VMEM((N_DEV, CH, D_MODEL), jnp.bfloat16),
            pltpu.VMEM((N_DEV, N_DEV - 1, CH, QC), jnp.bfloat16),
            pltpu.VMEM((N_DEV, CH, QC), jnp.bfloat16),
            pltpu.VMEM((N_DEV, N_DEV - 1, CH, QC), jnp.bfloat16),
            pltpu.SemaphoreType.DMA((N_DEV * 3,)),
            pltpu.SemaphoreType.DMA((N_DEV * 3,)),
            pltpu.SemaphoreType.DMA((N_DEV * 3,)),
            pltpu.SemaphoreType.DMA((N_DEV * 3,)),
        ],
        compiler_params=pltpu.CompilerParams(
            collective_id=0,
            vmem_limit_bytes=100 * 1024 * 1024,
        ),
    )(xs, Wq_l, K2, V2, Wo_l)
    return out[None]


# device time: 116363 ns/iter; 2.3750x vs baseline; 1.0203x over previous
import jax
import jax.numpy as jnp
from jax import lax
from jax.experimental import pallas as pl
from jax.experimental.pallas import tpu as pltpu

N_DEV = 4
SQ = 2048
SKV = 2048
HQ_LOCAL = 8
DH = 128
D_MODEL = 1024
D_LOCAL = HQ_LOCAL * DH
SCALE = 0.08838834764831843
BLK = 64
CH = SQ // N_DEV
QC = D_MODEL // N_DEV


def kernel(x, Wq, K_ext, V_ext, Wo):
    my = lax.axis_index("i")
    xs = x[0].astype(jnp.bfloat16)
    Wq_l = lax.dynamic_slice_in_dim(Wq, my * D_LOCAL, D_LOCAL, axis=1)
    Wq_l = Wq_l.astype(jnp.bfloat16)
    Wo_l = lax.dynamic_slice_in_dim(Wo, my * D_LOCAL, D_LOCAL, axis=0)
    Wo_l = Wo_l.astype(jnp.bfloat16)
    K2 = K_ext[0].reshape(SKV, D_LOCAL).astype(jnp.bfloat16)
    V2 = V_ext[0].reshape(SKV, D_LOCAL).astype(jnp.bfloat16)

    def body(x_ref, wq_ref, k_ref, v_ref, wo_ref, out_ref,
             ctx_ref, q_ref, rs_send_ref, rs_recv_ref, ag_send_ref,
             ag_recv_ref, rs_send_sems, rs_recv_sems, ag_send_sems,
             ag_recv_sems):
        my_pos = lax.axis_index("i")
        peers = [lax.rem(my_pos + 1 + p, N_DEV) for p in range(N_DEV - 1)]

        barrier_sem = pltpu.get_barrier_semaphore()
        for pr in peers:
            pl.semaphore_signal(
                barrier_sem, inc=1,
                device_id=(pr,), device_id_type=pl.DeviceIdType.MESH,
            )
        pl.semaphore_wait(barrier_sem, N_DEV - 1)

        my_cols = pl.ds(my_pos * QC, QC)
        pending_sends = []

        for r in range(N_DEV):
            rows = pl.ds(r * CH, CH)
            nk = (r + 1) * CH
            q_all = jnp.dot(x_ref[rows, :], wq_ref[...],
                            preferred_element_type=jnp.float32)
            q_ref[...] = (q_all * SCALE).astype(jnp.bfloat16)
            qb = lax.broadcasted_iota(jnp.int32, (CH, 1), 0) // BLK \
                + r * (CH // BLK)
            kb = lax.broadcasted_iota(jnp.int32, (1, nk), 1) // BLK
            keep = kb <= qb

            def head_body(h, carry, r=r, nk=nk, keep=keep):
                c = pl.ds(h * DH, DH)
                s = lax.dot_general(q_ref[:, c], k_ref[pl.ds(0, nk), c],
                                    (((1,), (1,)), ((), ())),
                                    preferred_element_type=jnp.float32)
                w = jnp.exp(jnp.where(keep, s, -30.0))
                denom = jnp.sum(w, axis=1, keepdims=True)
                ctx = jnp.dot(w.astype(jnp.bfloat16), v_ref[pl.ds(0, nk), c],
                              preferred_element_type=jnp.float32)
                ctx_ref[pl.ds(r * CH, CH), c] = (ctx / denom).astype(
                    jnp.bfloat16)
                return carry

            lax.fori_loop(0, HQ_LOCAL, head_body, 0, unroll=4)
            partial_r = jnp.dot(ctx_ref[rows, :], wo_ref[...],
                                preferred_element_type=jnp.float32)
            out_ref[rows, :] = partial_r
            rs_send_ref[r, :, :] = partial_r.astype(jnp.bfloat16)

            for p in range(N_DEV - 1):
                tgt = peers[p]
                rdma = pltpu.make_async_remote_copy(
                    src_ref=rs_send_ref.at[r, :, pl.ds(tgt * QC, QC)],
                    dst_ref=rs_recv_ref.at[r, 2 - p],
                    send_sem=rs_send_sems.at[r * 3 + p],
                    recv_sem=rs_recv_sems.at[r * 3 + (2 - p)],
                    device_id=(tgt,),
                    device_id_type=pl.DeviceIdType.MESH,
                )
                rdma.start()
                pending_sends.append(rdma)

            acc = out_ref[rows, my_cols]
            for q in range(N_DEV - 1):
                recv = pltpu.make_async_remote_copy(
                    src_ref=rs_recv_ref.at[r, q],
                    dst_ref=rs_recv_ref.at[r, q],
                    send_sem=rs_send_sems.at[r * 3 + q],
                    recv_sem=rs_recv_sems.at[r * 3 + q],
                    device_id=(my_pos,),
                    device_id_type=pl.DeviceIdType.MESH,
                )
                recv.wait_recv()
                acc = acc + rs_recv_ref[r, q].astype(jnp.float32)
            out_ref[rows, my_cols] = acc
            ag_send_ref[r, :, :] = acc.astype(jnp.bfloat16)
            for p in range(N_DEV - 1):
                tgt = peers[p]
                rdma = pltpu.make_async_remote_copy(
                    src_ref=ag_send_ref.at[r],
                    dst_ref=ag_recv_ref.at[r, 2 - p],
                    send_sem=ag_send_sems.at[r * 3 + p],
                    recv_sem=ag_recv_sems.at[r * 3 + (2 - p)],
                    device_id=(tgt,),
                    device_id_type=pl.DeviceIdType.MESH,
                )
                rdma.start()
                pending_sends.append(rdma)

        for r in range(N_DEV):
            rows = pl.ds(r * CH, CH)
            for q in range(N_DEV - 1):
                recv = pltpu.make_async_remote_copy(
                    src_ref=ag_recv_ref.at[r, q],
                    dst_ref=ag_recv_ref.at[r, q],
                    send_sem=ag_send_sems.at[r * 3 + q],
                    recv_sem=ag_recv_sems.at[r * 3 + q],
                    device_id=(my_pos,),
                    device_id_type=pl.DeviceIdType.MESH,
                )
                recv.wait_recv()
                out_ref[rows, pl.ds(peers[q] * QC, QC)] = (
                    ag_recv_ref[r, q].astype(jnp.float32))

        for rdma in pending_sends:
            rdma.wait_send()

    out = pl.pallas_call(
        body,
        out_shape=jax.ShapeDtypeStruct((SQ, D_MODEL), jnp.float32),
        in_specs=[pl.BlockSpec(memory_space=pltpu.VMEM)] * 5,
        out_specs=pl.BlockSpec(memory_space=pltpu.VMEM),
        scratch_shapes=[
            pltpu.VMEM((SQ, D_LOCAL), jnp.bfloat16),
            pltpu.VMEM((CH, D_LOCAL), jnp.bfloat16),
            pltpu.VMEM((N_DEV, CH, D_MODEL), jnp.bfloat16),
            pltpu.VMEM((N_DEV, N_DEV - 1, CH, QC), jnp.bfloat16),
            pltpu.VMEM((N_DEV, CH, QC), jnp.bfloat16),
            pltpu.VMEM((N_DEV, N_DEV - 1, CH, QC), jnp.bfloat16),
            pltpu.SemaphoreType.DMA((N_DEV * 3,)),
            pltpu.SemaphoreType.DMA((N_DEV * 3,)),
            pltpu.SemaphoreType.DMA((N_DEV * 3,)),
            pltpu.SemaphoreType.DMA((N_DEV * 3,)),
        ],
        compiler_params=pltpu.CompilerParams(
            collective_id=0,
            vmem_limit_bytes=100 * 1024 * 1024,
        ),
    )(xs, Wq_l, K2, V2, Wo_l)
    return out[None]


# device time: 95735 ns/iter; 2.8867x vs baseline; 1.2155x over previous
import jax
import jax.numpy as jnp
from jax import lax
from jax.experimental import pallas as pl
from jax.experimental.pallas import tpu as pltpu

N_DEV = 4
SQ = 2048
SKV = 2048
HQ_LOCAL = 8
DH = 128
D_MODEL = 1024
D_LOCAL = HQ_LOCAL * DH
SCALE = 0.08838834764831843
BLK = 64
CH = SQ // N_DEV
QC = D_MODEL // N_DEV


def kernel(x, Wq, K_ext, V_ext, Wo):
    my = lax.axis_index("i")
    xs = x[0].astype(jnp.bfloat16)
    Wq_l = lax.dynamic_slice_in_dim(Wq, my * D_LOCAL, D_LOCAL, axis=1)
    Wq_l = Wq_l.astype(jnp.bfloat16)
    Wo_l = lax.dynamic_slice_in_dim(Wo, my * D_LOCAL, D_LOCAL, axis=0)
    Wo_l = Wo_l.astype(jnp.bfloat16)
    K2 = K_ext[0].reshape(SKV, D_LOCAL).astype(jnp.bfloat16)
    V2 = V_ext[0].reshape(SKV, D_LOCAL).astype(jnp.bfloat16)

    def body(x_ref, wq_ref, k_ref, v_ref, wo_ref, out_ref,
             ctx_ref, q_ref, rs_send_ref, rs_recv_ref, ag_send_ref,
             ag_recv_ref, rs_send_sems, rs_recv_sems, ag_send_sems,
             ag_recv_sems):
        my_pos = lax.axis_index("i")
        peers = [lax.rem(my_pos + 1 + p, N_DEV) for p in range(N_DEV - 1)]

        barrier_sem = pltpu.get_barrier_semaphore()
        for pr in peers:
            pl.semaphore_signal(
                barrier_sem, inc=1,
                device_id=(pr,), device_id_type=pl.DeviceIdType.MESH,
            )
        pl.semaphore_wait(barrier_sem, N_DEV - 1)

        my_cols = pl.ds(my_pos * QC, QC)
        pending_sends = []

        def reduce_and_broadcast(r):
            rows = pl.ds(r * CH, CH)
            acc = out_ref[rows, my_cols]
            for q in range(N_DEV - 1):
                recv = pltpu.make_async_remote_copy(
                    src_ref=rs_recv_ref.at[r, q],
                    dst_ref=rs_recv_ref.at[r, q],
                    send_sem=rs_send_sems.at[r * 3 + q],
                    recv_sem=rs_recv_sems.at[r * 3 + q],
                    device_id=(my_pos,),
                    device_id_type=pl.DeviceIdType.MESH,
                )
                recv.wait_recv()
                acc = acc + rs_recv_ref[r, q].astype(jnp.float32)
            out_ref[rows, my_cols] = acc
            ag_send_ref[r, :, :] = acc.astype(jnp.bfloat16)
            for p in range(N_DEV - 1):
                tgt = peers[p]
                rdma = pltpu.make_async_remote_copy(
                    src_ref=ag_send_ref.at[r],
                    dst_ref=ag_recv_ref.at[r, 2 - p],
                    send_sem=ag_send_sems.at[r * 3 + p],
                    recv_sem=ag_recv_sems.at[r * 3 + (2 - p)],
                    device_id=(tgt,),
                    device_id_type=pl.DeviceIdType.MESH,
                )
                rdma.start()
                pending_sends.append(rdma)

        for r in range(N_DEV):
            rows = pl.ds(r * CH, CH)
            nk = (r + 1) * CH
            q_all = jnp.dot(x_ref[rows, :], wq_ref[...],
                            preferred_element_type=jnp.float32)
            q_ref[...] = (q_all * SCALE).astype(jnp.bfloat16)
            qb = lax.broadcasted_iota(jnp.int32, (CH, 1), 0) // BLK \
                + r * (CH // BLK)
            kb = lax.broadcasted_iota(jnp.int32, (1, nk), 1) // BLK
            keep = kb <= qb

            def head_body(h, carry, r=r, nk=nk, keep=keep):
                c = pl.ds(h * DH, DH)
                s = lax.dot_general(q_ref[:, c], k_ref[pl.ds(0, nk), c],
                                    (((1,), (1,)), ((), ())),
                                    preferred_element_type=jnp.float32)
                w = jnp.exp(jnp.where(keep, s, -30.0))
                denom = jnp.sum(w, axis=1, keepdims=True)
                ctx = jnp.dot(w.astype(jnp.bfloat16), v_ref[pl.ds(0, nk), c],
                              preferred_element_type=jnp.float32)
                ctx_ref[pl.ds(r * CH, CH), c] = (ctx / denom).astype(
                    jnp.bfloat16)
                return carry

            lax.fori_loop(0, HQ_LOCAL, head_body, 0, unroll=4)
            partial_r = jnp.dot(ctx_ref[rows, :], wo_ref[...],
                                preferred_element_type=jnp.float32)
            out_ref[rows, :] = partial_r
            rs_send_ref[r, :, :] = partial_r.astype(jnp.bfloat16)

            for p in range(N_DEV - 1):
                tgt = peers[p]
                rdma = pltpu.make_async_remote_copy(
                    src_ref=rs_send_ref.at[r, :, pl.ds(tgt * QC, QC)],
                    dst_ref=rs_recv_ref.at[r, 2 - p],
                    send_sem=rs_send_sems.at[r * 3 + p],
                    recv_sem=rs_recv_sems.at[r * 3 + (2 - p)],
                    device_id=(tgt,),
                    device_id_type=pl.DeviceIdType.MESH,
                )
                rdma.start()
                pending_sends.append(rdma)

            if r >= 1:
                reduce_and_broadcast(r - 1)

        reduce_and_broadcast(N_DEV - 1)

        for r in range(N_DEV):
            rows = pl.ds(r * CH, CH)
            for q in range(N_DEV - 1):
                recv = pltpu.make_async_remote_copy(
                    src_ref=ag_recv_ref.at[r, q],
                    dst_ref=ag_recv_ref.at[r, q],
                    send_sem=ag_send_sems.at[r * 3 + q],
                    recv_sem=ag_recv_sems.at[r * 3 + q],
                    device_id=(my_pos,),
                    device_id_type=pl.DeviceIdType.MESH,
                )
                recv.wait_recv()
                out_ref[rows, pl.ds(peers[q] * QC, QC)] = (
                    ag_recv_ref[r, q].astype(jnp.float32))

        for rdma in pending_sends:
            rdma.wait_send()

    out = pl.pallas_call(
        body,
        out_shape=jax.ShapeDtypeStruct((SQ, D_MODEL), jnp.float32),
        in_specs=[pl.BlockSpec(memory_space=pltpu.VMEM)] * 5,
        out_specs=pl.BlockSpec(memory_space=pltpu.VMEM),
        scratch_shapes=[
            pltpu.VMEM((SQ, D_LOCAL), jnp.bfloat16),
            pltpu.VMEM((CH, D_LOCAL), jnp.bfloat16),
            pltpu.VMEM((N_DEV, CH, D_MODEL), jnp.bfloat16),
            pltpu.VMEM((N_DEV, N_DEV - 1, CH, QC), jnp.bfloat16),
            pltpu.VMEM((N_DEV, CH, QC), jnp.bfloat16),
            pltpu.VMEM((N_DEV, N_DEV - 1, CH, QC), jnp.bfloat16),
            pltpu.SemaphoreType.DMA((N_DEV * 3,)),
            pltpu.SemaphoreType.DMA((N_DEV * 3,)),
            pltpu.SemaphoreType.DMA((N_DEV * 3,)),
            pltpu.SemaphoreType.DMA((N_DEV * 3,)),
        ],
        compiler_params=pltpu.CompilerParams(
            collective_id=0,
            vmem_limit_bytes=100 * 1024 * 1024,
        ),
    )(xs, Wq_l, K2, V2, Wo_l)
    return out[None]
